# baseline (speedup 1.0000x reference)
import dataclasses
import functools

import jax
import jax.numpy as jnp
from jax import lax
from jax.experimental import pallas as pl
from jax.experimental.pallas import tpu as pltpu
from jax.experimental.pallas import tpu_sc as plsc

M = 4
B = 4096
N = 1600000
N3 = 3 * N


_CB = 48000
_GRID = N3 // _CB


def _dense_body(mf_ref, df_ref, fm_ref, sv_ref, ad_ref, sd_ref):
    r0 = mf_ref[0, :]
    r1 = mf_ref[1, :]
    r2 = mf_ref[2, :]
    r3 = mf_ref[3, :]
    mean = (r0 + r1 + r2 + r3) * 0.25
    d0 = r0 - mean
    d1 = r1 - mean
    d2 = r2 - mean
    d3 = r3 - mean
    var = (d0 * d0 + d1 * d1 + d2 * d2 + d3 * d3) * (1.0 / 3.0)
    diff = mean - df_ref[0, :]
    fm_ref[0, :] = mean
    sv_ref[0, :] = var
    ad_ref[0, :] = jnp.abs(diff)
    sd_ref[0, :] = diff * diff


def _dense_stage(mf2, df2):
    flat = jax.ShapeDtypeStruct((1, N3), jnp.float32)
    return pl.pallas_call(
        _dense_body,
        grid=(_GRID,),
        in_specs=[
            pl.BlockSpec((M, _CB), lambda j: (0, j)),
            pl.BlockSpec((1, _CB), lambda j: (0, j)),
        ],
        out_specs=[pl.BlockSpec((1, _CB), lambda j: (0, j))] * 4,
        out_shape=[flat, flat, flat, flat],
    )(mf2, df2)


_NW = 32
_PW = N // _NW
_CH = 2000
_NCH = _PW // _CH
_NST = _CH // 16


def _take(x, idx):
    return lax.gather(
        x, idx[:, None],
        dimension_numbers=lax.GatherDimensionNumbers(
            offset_dims=(), collapsed_slice_dims=(0,), start_index_map=(0,)),
        slice_sizes=(1,),
        mode=lax.GatherScatterMode.PROMISE_IN_BOUNDS)


def _sc_body(ids_hbm, sv_hbm, ad_hbm, sd_hbm, out_hbm,
             ids_v, sv_v, ad_v, sd_v, svt, adt, sdt, cntt, maxt, mint):
    wid = lax.axis_index("s") * 2 + lax.axis_index("c")

    @pl.loop(0, B, step=16)
    def _init(k):
        z = jnp.zeros((16,), jnp.float32)
        svt[pl.ds(k, 16)] = z
        adt[pl.ds(k, 16)] = z
        sdt[pl.ds(k, 16)] = z
        cntt[pl.ds(k, 16)] = z
        maxt[pl.ds(k, 16)] = jnp.full((16,), -jnp.inf, jnp.float32)
        mint[pl.ds(k, 16)] = jnp.full((16,), jnp.inf, jnp.float32)

    iota = lax.iota(jnp.int32, 16)
    i3 = iota * 3
    ones = jnp.ones((16,), jnp.float32)
    last15 = iota == 15
    nxt = jnp.minimum(iota + 1, 15)
    shifts = [(d, jnp.maximum(iota - d, 0)) for d in (1, 2, 4, 8)]

    @pl.loop(0, _NCH)
    def _chunk(ci):
        base = wid * _PW + ci * _CH
        pltpu.sync_copy(ids_hbm.at[pl.ds(base, _CH)], ids_v)
        pltpu.sync_copy(sv_hbm.at[pl.ds(3 * base, 3 * _CH)], sv_v)
        pltpu.sync_copy(ad_hbm.at[pl.ds(3 * base, 3 * _CH)], ad_v)
        pltpu.sync_copy(sd_hbm.at[pl.ds(3 * base, 3 * _CH)], sd_v)

        @pl.loop(0, _NST)
        def _step(st):
            g = ids_v[pl.ds(st * 16, 16)]
            idx0 = i3 + st * 48

            def csum(ref):
                return (plsc.load_gather(ref, [idx0])
                        + plsc.load_gather(ref, [idx0 + 1])
                        + plsc.load_gather(ref, [idx0 + 2]))

            s_sv = csum(sv_v)
            s_ad = csum(ad_v)
            s_sd = csum(sd_v)
            plsc.addupdate_scatter(svt, [g], s_sv)
            plsc.addupdate_scatter(adt, [g], s_ad)
            plsc.addupdate_scatter(sdt, [g], s_sd)
            plsc.addupdate_scatter(cntt, [g], ones)

            mx = s_sd
            mn = s_sd
            for _, idxd in shifts:
                same = _take(g, idxd) == g
                mx = jnp.where(same, jnp.maximum(mx, _take(mx, idxd)), mx)
                mn = jnp.where(same, jnp.minimum(mn, _take(mn, idxd)), mn)
            lastocc = (g != _take(g, nxt)) | last15
            cur_mx = plsc.load_gather(maxt, [g])
            cur_mn = plsc.load_gather(mint, [g])
            plsc.store_scatter(maxt, [g], jnp.maximum(cur_mx, mx), mask=lastocc)
            plsc.store_scatter(mint, [g], jnp.minimum(cur_mn, mn), mask=lastocc)

    pltpu.sync_copy(svt, out_hbm.at[0, wid])
    pltpu.sync_copy(adt, out_hbm.at[1, wid])
    pltpu.sync_copy(sdt, out_hbm.at[2, wid])
    pltpu.sync_copy(cntt, out_hbm.at[3, wid])
    pltpu.sync_copy(maxt, out_hbm.at[4, wid])
    pltpu.sync_copy(mint, out_hbm.at[5, wid])


def _segment_stage(image_idx, sv_flat, ad_flat, sd_flat):
    mesh = plsc.VectorSubcoreMesh(core_axis_name="c", subcore_axis_name="s")
    cp = pltpu.CompilerParams()
    if "needs_layout_passes" in pltpu.CompilerParams.__dataclass_fields__:
        cp = dataclasses.replace(cp, needs_layout_passes=False)
    fn = pl.kernel(
        _sc_body,
        out_type=jax.ShapeDtypeStruct((6, _NW, B), jnp.float32),
        mesh=mesh,
        scratch_types=[
            pltpu.VMEM((_CH,), jnp.int32),
            pltpu.VMEM((3 * _CH,), jnp.float32),
            pltpu.VMEM((3 * _CH,), jnp.float32),
            pltpu.VMEM((3 * _CH,), jnp.float32),
            pltpu.VMEM((B,), jnp.float32),
            pltpu.VMEM((B,), jnp.float32),
            pltpu.VMEM((B,), jnp.float32),
            pltpu.VMEM((B,), jnp.float32),
            pltpu.VMEM((B,), jnp.float32),
            pltpu.VMEM((B,), jnp.float32),
        ],
        compiler_params=cp,
    )
    return fn(image_idx, sv_flat, ad_flat, sd_flat)


def _combine_body(pt_ref, me_ref, de_ref,
                  emean_o, emax_o, emin_o, evar_o, esd_o, eae_o, ese_o,
                  fvar_o, fsd_o, fae_o, fse_o, fmaxe_o, fmine_o):
    pt = pt_ref[...]
    sv = jnp.sum(pt[0:_NW], axis=0)
    sa = jnp.sum(pt[_NW:2 * _NW], axis=0)
    ss = jnp.sum(pt[2 * _NW:3 * _NW], axis=0)
    cnt = jnp.sum(pt[3 * _NW:4 * _NW], axis=0)
    mx = jnp.max(pt[4 * _NW:5 * _NW], axis=0)
    mn = jnp.min(pt[5 * _NW:6 * _NW], axis=0)
    inv3 = 1.0 / (3.0 * jnp.maximum(cnt, 1.0))
    fvar = sv * inv3
    fvar_o[0, :] = fvar
    fsd_o[0, :] = jnp.sqrt(fvar)
    fae_o[0, :] = sa * inv3
    fse_o[0, :] = ss * inv3
    pos = cnt > 0.0
    fmaxe_o[0, :] = jnp.where(pos, jnp.sqrt(jnp.maximum(mx, 0.0)), -jnp.inf)
    fmine_o[0, :] = jnp.where(pos, jnp.sqrt(jnp.maximum(mn, 0.0)), jnp.inf)

    me = me_ref[...]
    emean = jnp.mean(me, axis=0)
    emean_o[0, :] = emean
    emax_o[...] = jnp.max(me).reshape(1, 1)
    emin_o[...] = jnp.min(me).reshape(1, 1)
    dev = me - emean[None, :]
    evar = jnp.sum(dev * dev, axis=0) * (1.0 / 3.0)
    evar_o[0, :] = evar
    esd_o[0, :] = jnp.sqrt(evar)
    ediff = emean - de_ref[0, :]
    eae_o[0, :] = jnp.abs(ediff)
    ese_o[0, :] = ediff * ediff


def _combine_stage(partials2, me, de2):
    vb = jax.ShapeDtypeStruct((1, B), jnp.float32)
    s1 = jax.ShapeDtypeStruct((1, 1), jnp.float32)
    return pl.pallas_call(
        _combine_body,
        out_shape=[vb, s1, s1, vb, vb, vb, vb, vb, vb, vb, vb, vb, vb],
    )(partials2, me, de2)


def kernel(model_energies, model_forces, data_energy, data_forces, image_idx):
    mf2 = model_forces.reshape(M, N3)
    df2 = data_forces.reshape(1, N3)

    fm, sv_flat, ad_flat, sd_flat = _dense_stage(mf2, df2)

    partials = _segment_stage(image_idx,
                              sv_flat.reshape(N3), ad_flat.reshape(N3),
                              sd_flat.reshape(N3))

    (emean, emax, emin, evar, esd, eae, ese,
     fvar, fsd, fae, fse, fmaxe, fmine) = _combine_stage(
        partials.reshape(6 * _NW, B), model_energies,
        data_energy.reshape(1, B))

    return (emean.reshape(B), fm.reshape(N, 3), emax.reshape(1),
            emin.reshape(1), evar.reshape(B), esd.reshape(B),
            fvar.reshape(B), fsd.reshape(B), eae.reshape(B),
            ese.reshape(B), fae.reshape(B), fse.reshape(B),
            fmaxe.reshape(B), fmine.reshape(B))


# speedup vs baseline: 48.2396x; 48.2396x over previous
import dataclasses

import jax
import jax.numpy as jnp
from jax import lax
from jax.experimental import pallas as pl
from jax.experimental.pallas import tpu as pltpu
from jax.experimental.pallas import tpu_sc as plsc

M = 4
B = 4096
N = 1600000


_CB = 64000
_GRID_D = N // _CB


def _dense_body(mf_ref, df_ref, fm_ref, sv_ref, ad_ref, sd_ref):
    mm = [mf_ref[:, m, :] for m in range(4)]
    mean3 = (mm[0] + mm[1] + mm[2] + mm[3]) * 0.25
    var3 = sum((x - mean3) * (x - mean3) for x in mm) * (1.0 / 3.0)
    diff3 = mean3 - df_ref[...]
    fm_ref[...] = mean3
    sv_ref[0, :] = jnp.sum(var3, axis=0)
    ad_ref[0, :] = jnp.sum(jnp.abs(diff3), axis=0)
    sd_ref[0, :] = jnp.sum(diff3 * diff3, axis=0)


def _dense_stage(mfw, dfw):
    stat = jax.ShapeDtypeStruct((1, N), jnp.float32)
    return pl.pallas_call(
        _dense_body,
        grid=(_GRID_D,),
        in_specs=[
            pl.BlockSpec((3, 4, _CB), lambda j: (0, 0, j)),
            pl.BlockSpec((3, _CB), lambda j: (0, j)),
        ],
        out_specs=[
            pl.BlockSpec((3, _CB), lambda j: (0, j)),
            pl.BlockSpec((1, _CB), lambda j: (0, j)),
            pl.BlockSpec((1, _CB), lambda j: (0, j)),
            pl.BlockSpec((1, _CB), lambda j: (0, j)),
        ],
        out_shape=[jax.ShapeDtypeStruct((3, N), jnp.float32),
                   stat, stat, stat],
    )(mfw, dfw)


_NW = 32
_PW = N // _NW
_CH = 2000
_NCH = _PW // _CH
_NST = _CH // 16


def _take(x, idx):
    return lax.gather(
        x, idx[:, None],
        dimension_numbers=lax.GatherDimensionNumbers(
            offset_dims=(), collapsed_slice_dims=(0,), start_index_map=(0,)),
        slice_sizes=(1,),
        mode=lax.GatherScatterMode.PROMISE_IN_BOUNDS)


def _sc_body(ids_hbm, sv_hbm, ad_hbm, sd_hbm, out_hbm,
             ids_v, sv_v, ad_v, sd_v, svt, adt, sdt, cntt, maxt, mint):
    wid = lax.axis_index("s") * 2 + lax.axis_index("c")

    @pl.loop(0, B, step=16)
    def _init(k):
        z = jnp.zeros((16,), jnp.float32)
        svt[pl.ds(k, 16)] = z
        adt[pl.ds(k, 16)] = z
        sdt[pl.ds(k, 16)] = z
        cntt[pl.ds(k, 16)] = z
        maxt[pl.ds(k, 16)] = jnp.full((16,), -jnp.inf, jnp.float32)
        mint[pl.ds(k, 16)] = jnp.full((16,), jnp.inf, jnp.float32)

    iota = lax.iota(jnp.int32, 16)
    ones = jnp.ones((16,), jnp.float32)
    last15 = iota == 15
    nxt = jnp.minimum(iota + 1, 15)
    shifts = [jnp.maximum(iota - d, 0) for d in (1, 2, 4, 8)]

    @pl.loop(0, _NCH)
    def _chunk(ci):
        base = wid * _PW + ci * _CH
        pltpu.sync_copy(ids_hbm.at[pl.ds(base, _CH)], ids_v)
        pltpu.sync_copy(sv_hbm.at[pl.ds(base, _CH)], sv_v)
        pltpu.sync_copy(ad_hbm.at[pl.ds(base, _CH)], ad_v)
        pltpu.sync_copy(sd_hbm.at[pl.ds(base, _CH)], sd_v)

        @pl.loop(0, _NST)
        def _step(st):
            g = ids_v[pl.ds(st * 16, 16)]
            s_sv = sv_v[pl.ds(st * 16, 16)]
            s_ad = ad_v[pl.ds(st * 16, 16)]
            s_sd = sd_v[pl.ds(st * 16, 16)]
            plsc.addupdate_scatter(svt, [g], s_sv)
            plsc.addupdate_scatter(adt, [g], s_ad)
            plsc.addupdate_scatter(sdt, [g], s_sd)
            plsc.addupdate_scatter(cntt, [g], ones)

            mx = s_sd
            mn = s_sd
            for idxd in shifts:
                same = _take(g, idxd) == g
                mx = jnp.where(same, jnp.maximum(mx, _take(mx, idxd)), mx)
                mn = jnp.where(same, jnp.minimum(mn, _take(mn, idxd)), mn)
            lastocc = (g != _take(g, nxt)) | last15
            cur_mx = plsc.load_gather(maxt, [g])
            cur_mn = plsc.load_gather(mint, [g])
            plsc.store_scatter(maxt, [g], jnp.maximum(cur_mx, mx), mask=lastocc)
            plsc.store_scatter(mint, [g], jnp.minimum(cur_mn, mn), mask=lastocc)

    pltpu.sync_copy(svt, out_hbm.at[0, wid])
    pltpu.sync_copy(adt, out_hbm.at[1, wid])
    pltpu.sync_copy(sdt, out_hbm.at[2, wid])
    pltpu.sync_copy(cntt, out_hbm.at[3, wid])
    pltpu.sync_copy(maxt, out_hbm.at[4, wid])
    pltpu.sync_copy(mint, out_hbm.at[5, wid])


def _segment_stage(image_idx, sv_flat, ad_flat, sd_flat):
    mesh = plsc.VectorSubcoreMesh(core_axis_name="c", subcore_axis_name="s")
    cp = pltpu.CompilerParams()
    if "needs_layout_passes" in pltpu.CompilerParams.__dataclass_fields__:
        cp = dataclasses.replace(cp, needs_layout_passes=False)
    fn = pl.kernel(
        _sc_body,
        out_type=jax.ShapeDtypeStruct((6, _NW, B), jnp.float32),
        mesh=mesh,
        scratch_types=[
            pltpu.VMEM((_CH,), jnp.int32),
            pltpu.VMEM((_CH,), jnp.float32),
            pltpu.VMEM((_CH,), jnp.float32),
            pltpu.VMEM((_CH,), jnp.float32),
            pltpu.VMEM((B,), jnp.float32),
            pltpu.VMEM((B,), jnp.float32),
            pltpu.VMEM((B,), jnp.float32),
            pltpu.VMEM((B,), jnp.float32),
            pltpu.VMEM((B,), jnp.float32),
            pltpu.VMEM((B,), jnp.float32),
        ],
        compiler_params=cp,
    )
    return fn(image_idx, sv_flat, ad_flat, sd_flat)


def _combine_body(pt_ref, me_ref, de_ref,
                  emean_o, emax_o, emin_o, evar_o, esd_o, eae_o, ese_o,
                  fvar_o, fsd_o, fae_o, fse_o, fmaxe_o, fmine_o):
    pt = pt_ref[...]
    sv = jnp.sum(pt[0:_NW], axis=0)
    sa = jnp.sum(pt[_NW:2 * _NW], axis=0)
    ss = jnp.sum(pt[2 * _NW:3 * _NW], axis=0)
    cnt = jnp.sum(pt[3 * _NW:4 * _NW], axis=0)
    mx = jnp.max(pt[4 * _NW:5 * _NW], axis=0)
    mn = jnp.min(pt[5 * _NW:6 * _NW], axis=0)
    inv3 = 1.0 / (3.0 * jnp.maximum(cnt, 1.0))
    fvar = sv * inv3
    fvar_o[0, :] = fvar
    fsd_o[0, :] = jnp.sqrt(fvar)
    fae_o[0, :] = sa * inv3
    fse_o[0, :] = ss * inv3
    pos = cnt > 0.0
    fmaxe_o[0, :] = jnp.where(pos, jnp.sqrt(jnp.maximum(mx, 0.0)), -jnp.inf)
    fmine_o[0, :] = jnp.where(pos, jnp.sqrt(jnp.maximum(mn, 0.0)), jnp.inf)

    me = me_ref[...]
    emean = jnp.mean(me, axis=0)
    emean_o[0, :] = emean
    emax_o[...] = jnp.max(me).reshape(1, 1)
    emin_o[...] = jnp.min(me).reshape(1, 1)
    dev = me - emean[None, :]
    evar = jnp.sum(dev * dev, axis=0) * (1.0 / 3.0)
    evar_o[0, :] = evar
    esd_o[0, :] = jnp.sqrt(evar)
    ediff = emean - de_ref[0, :]
    eae_o[0, :] = jnp.abs(ediff)
    ese_o[0, :] = ediff * ediff


def _combine_stage(partials2, me, de2):
    vb = jax.ShapeDtypeStruct((1, B), jnp.float32)
    s1 = jax.ShapeDtypeStruct((1, 1), jnp.float32)
    return pl.pallas_call(
        _combine_body,
        out_shape=[vb, s1, s1, vb, vb, vb, vb, vb, vb, vb, vb, vb, vb],
    )(partials2, me, de2)


def kernel(model_energies, model_forces, data_energy, data_forces, image_idx):
    mfw = model_forces.transpose(2, 0, 1)
    dfw = data_forces.transpose(1, 0)

    fm, sv, ad, sd = _dense_stage(mfw, dfw)

    partials = _segment_stage(image_idx, sv.reshape(N), ad.reshape(N),
                              sd.reshape(N))

    (emean, emax, emin, evar, esd, eae, ese,
     fvar, fsd, fae, fse, fmaxe, fmine) = _combine_stage(
        partials.reshape(6 * _NW, B), model_energies,
        data_energy.reshape(1, B))

    return (emean.reshape(B), fm.transpose(1, 0), emax.reshape(1),
            emin.reshape(1), evar.reshape(B), esd.reshape(B),
            fvar.reshape(B), fsd.reshape(B), eae.reshape(B),
            ese.reshape(B), fae.reshape(B), fse.reshape(B),
            fmaxe.reshape(B), fmine.reshape(B))
